# baseline (speedup 1.0000x reference)
import functools

import jax
import jax.numpy as jnp
import numpy as np
from jax import lax
from jax.experimental import pallas as pl
from jax.experimental.pallas import tpu as pltpu
from jax.experimental.pallas import tpu_sc as plsc

N_NODES = 10000
N_EDGES = 320000
D = 128
D_EDGE = 16
NC = 2
NS = 16
NW = NC * NS
EC = 40
NCHUNK = N_EDGES // EC
CPW = NCHUNK // NW
NROW_CHK = 400
NROW_NCHK = N_NODES // NROW_CHK
BE = 8000

f32 = jnp.float32
bf16 = jnp.bfloat16
DW = D // 2

_PERM_NP = np.zeros(D, dtype=np.int32)
for _g in range(D // 32):
    for _k in range(16):
        _PERM_NP[32 * _g + _k] = 32 * _g + 2 * _k
        _PERM_NP[32 * _g + 16 + _k] = 32 * _g + 2 * _k + 1
_P_NP = np.zeros((D, D), dtype=np.float32)
_P_NP[np.arange(D), _PERM_NP] = 1.0


def _mesh():
    return plsc.VectorSubcoreMesh(core_axis_name="c", subcore_axis_name="s",
                                  num_cores=NC, num_subcores=NS)


def _build_sc_pass(has_g: bool, write_r: bool):
    acc_t = jax.ShapeDtypeStruct((NC, N_NODES, D), f32)
    out_type = ([jax.ShapeDtypeStruct((N_EDGES * DW,), jnp.int32), acc_t]
                if write_r else acc_t)

    scratch = [pltpu.VMEM((EC,), jnp.int32) for _ in range(2)]
    if has_g:
        scratch += [pltpu.VMEM((EC,), jnp.int32) for _ in range(2)]
    scratch += [pltpu.VMEM((EC,), jnp.int32) for _ in range(2)]
    scratch += [pltpu.VMEM((EC * DW,), jnp.int32) for _ in range(2)]
    scratch += [pltpu.VMEM((EC, D), f32) for _ in range(2)]
    if has_g:
        scratch += [pltpu.VMEM((EC, D), f32) for _ in range(2)]
    scratch.append(pltpu.VMEM_SHARED((N_NODES, D), f32))
    scratch += [pltpu.SemaphoreType.DMA for _ in range(8)]

    @functools.partial(pl.kernel, out_type=out_type, mesh=_mesh(),
                       scratch_types=scratch)
    def k(*args):
        it = iter(args)
        ep = next(it)
        qt = next(it)
        gt = next(it) if has_g else None
        src = next(it)
        rev = next(it) if has_g else None
        dst = next(it)
        zeros = next(it)
        r_out = next(it) if write_r else None
        acc = next(it)
        sibuf = [next(it), next(it)]
        ribuf = [next(it), next(it)] if has_g else None
        dbuf = [next(it), next(it)]
        eprow = [next(it), next(it)]
        qrow = [next(it), next(it)]
        grow = [next(it), next(it)] if has_g else None
        shacc = next(it)
        sem_idx = [next(it), next(it)]
        sem_rows = [next(it), next(it)]
        sem_sc = [next(it), next(it)]
        sem_st = [next(it), next(it)]

        cid = lax.axis_index("c")
        sid = lax.axis_index("s")
        wid = sid * NC + cid

        nlo = sid * NROW_NCHK // NS
        nhi = (sid + 1) * NROW_NCHK // NS

        def zero_chunk(kk, carry):
            rr = kk * NROW_CHK
            pltpu.sync_copy(zeros.at[pl.ds(rr, NROW_CHK)],
                            shacc.at[pl.ds(rr, NROW_CHK)])
            return carry

        lax.fori_loop(nlo, nhi, zero_chunk, 0)
        plsc.subcore_barrier()

        ebase = wid * CPW

        def issue_gidx(t, p):
            b = (ebase + t) * EC
            pltpu.async_copy(src.at[pl.ds(b, EC)], sibuf[p], sem_idx[p])
            if has_g:
                pltpu.async_copy(rev.at[pl.ds(b, EC)], ribuf[p], sem_idx[p])

        def issue_didx(t, p):
            b = (ebase + t) * EC
            pltpu.async_copy(dst.at[pl.ds(b, EC)], dbuf[p], sem_idx[p])

        def wait_idx(p):
            pltpu.make_async_copy(src.at[pl.ds(0, EC)], sibuf[p],
                                  sem_idx[p]).wait()
            if has_g:
                pltpu.make_async_copy(src.at[pl.ds(0, EC)], ribuf[p],
                                      sem_idx[p]).wait()
            pltpu.make_async_copy(src.at[pl.ds(0, EC)], dbuf[p],
                                  sem_idx[p]).wait()

        def issue_rows(t, p):
            b = (ebase + t) * EC
            de = pltpu.async_copy(ep.at[pl.ds(b * DW, EC * DW)], eprow[p],
                                  sem_rows[p])
            dq = pltpu.async_copy(qt.at[sibuf[p]], qrow[p], sem_rows[p])
            dg = (pltpu.async_copy(gt.at[ribuf[p]], grow[p], sem_rows[p])
                  if has_g else None)
            return de, dq, dg

        def compute(p):
            himask = jnp.int32(-65536)
            half = jnp.int32(0x8000)

            @plsc.parallel_loop(0, EC, 1, unroll=2)
            def row(e):
                for g in range(D // 32):
                    slw = pl.ds(e * DW + g * 16, 16)
                    sle = pl.ds(g * 32, 16)
                    slo = pl.ds(g * 32 + 16, 16)
                    xi = eprow[p][slw]
                    ee = lax.bitcast_convert_type(lax.shift_left(xi, 16), f32)
                    eo = lax.bitcast_convert_type(xi & himask, f32)
                    ve = ee + qrow[p][e, sle]
                    vo = eo + qrow[p][e, slo]
                    if has_g:
                        ve = ve - grow[p][e, sle]
                        vo = vo - grow[p][e, slo]
                    ve = jnp.maximum(ve, 0.0)
                    vo = jnp.maximum(vo, 0.0)
                    qrow[p][e, sle] = ve
                    qrow[p][e, slo] = vo
                    if write_r:
                        ie = lax.bitcast_convert_type(ve, jnp.int32) + half
                        io = lax.bitcast_convert_type(vo, jnp.int32) + half
                        eprow[p][slw] = ((io & himask)
                                         | lax.shift_right_logical(ie, 16))

        def issue_outs(t, p):
            dsc = pltpu.async_copy(qrow[p], shacc.at[dbuf[p]], sem_sc[p],
                                   add=True)
            if write_r:
                b = (ebase + t) * EC
                pltpu.async_copy(eprow[p], r_out.at[pl.ds(b * DW, EC * DW)],
                                 sem_st[p])
            return dsc

        def wait_store(p):
            if write_r:
                pltpu.make_async_copy(ep.at[pl.ds(0, EC * DW)], eprow[p],
                                      sem_st[p]).wait()

        def finish(t, p):
            compute(p)
            dsc = issue_outs(t, p)
            return dsc

        b0 = ebase * EC
        pltpu.sync_copy(src.at[pl.ds(b0, EC)], sibuf[0])
        if has_g:
            pltpu.sync_copy(rev.at[pl.ds(b0, EC)], ribuf[0])
        pltpu.sync_copy(dst.at[pl.ds(b0, EC)], dbuf[0])
        de, dq, dg = issue_rows(0, 0)
        issue_gidx(1, 1)
        de.wait()
        dq.wait()
        if has_g:
            dg.wait()
        issue_didx(1, 1)
        wait_idx(1)
        de1, dq1, dg1 = issue_rows(1, 1)
        issue_gidx(2, 0)
        dsc0 = finish(0, 0)
        dsc0.wait()
        de1.wait()
        dq1.wait()
        if has_g:
            dg1.wait()
        issue_didx(2, 0)

        def body(m, carry):
            t0 = 2 * m
            t1 = t0 + 1

            wait_idx(0)
            wait_store(0)
            de, dq, dg = issue_rows(t0, 0)
            issue_gidx(t0 + 1, 1)
            dsc = finish(t0 - 1, 1)
            de.wait()
            dq.wait()
            if has_g:
                dg.wait()
            dsc.wait()
            issue_didx(t0 + 1, 1)

            wait_idx(1)
            wait_store(1)
            de1, dq1, dg1 = issue_rows(t1, 1)

            @pl.when(m < CPW // 2 - 1)
            def _i1():
                issue_gidx(t1 + 1, 0)

            dsc0 = finish(t0, 0)
            de1.wait()
            dq1.wait()
            if has_g:
                dg1.wait()
            dsc0.wait()

            @pl.when(m < CPW // 2 - 1)
            def _i2():
                issue_didx(t1 + 1, 0)

            return carry

        lax.fori_loop(1, CPW // 2, body, 0)

        dsc = finish(CPW - 1, 1)
        dsc.wait()
        wait_store(0)
        wait_store(1)

        plsc.subcore_barrier()

        def dump_chunk(kk, carry):
            rr = kk * NROW_CHK
            pltpu.sync_copy(shacc.at[pl.ds(rr, NROW_CHK)],
                            acc.at[cid, pl.ds(rr, NROW_CHK)])
            return carry

        lax.fori_loop(nlo, nhi, dump_chunk, 0)

    return k


_SC_BUILD = _build_sc_pass(has_g=False, write_r=True)
_SC_STEP = _build_sc_pass(has_g=True, write_r=True)
_SC_LAST = _build_sc_pass(has_g=True, write_r=False)


def _tc_xp(x, wt):
    def body(x_ref, w_ref, o_ref):
        o_ref[...] = jnp.dot(x_ref[...], w_ref[...], preferred_element_type=f32)

    return pl.pallas_call(
        body,
        out_shape=jax.ShapeDtypeStruct((N_NODES, D), f32),
        grid=(1,),
        in_specs=[pl.BlockSpec((N_NODES, D), lambda i: (0, 0)),
                  pl.BlockSpec((D, D), lambda i: (0, 0))],
        out_specs=pl.BlockSpec((N_NODES, D), lambda i: (0, 0)),
    )(x, wt)


def _tc_ep(ea, wt, b):
    def body(e_ref, w_ref, b_ref, o_ref):
        o_ref[...] = (jnp.dot(e_ref[...], w_ref[...], preferred_element_type=f32)
                      + b_ref[...]).astype(bf16)

    return pl.pallas_call(
        body,
        out_shape=jax.ShapeDtypeStruct((N_EDGES, D), bf16),
        grid=(N_EDGES // BE,),
        in_specs=[pl.BlockSpec((BE, D_EDGE), lambda i: (i, 0)),
                  pl.BlockSpec((D_EDGE, D), lambda i: (0, 0)),
                  pl.BlockSpec((1, D), lambda i: (0, 0))],
        out_specs=pl.BlockSpec((BE, D), lambda i: (i, 0)),
    )(ea, wt, b)


def _tc_g(r, wt):
    def body(r_ref, w_ref, o_ref):
        o_ref[...] = jnp.dot(r_ref[...], w_ref[...], preferred_element_type=f32)

    return pl.pallas_call(
        body,
        out_shape=jax.ShapeDtypeStruct((N_EDGES, D), f32),
        grid=(N_EDGES // BE,),
        in_specs=[pl.BlockSpec((BE, D), lambda i: (i, 0)),
                  pl.BlockSpec((D, D), lambda i: (0, 0))],
        out_specs=pl.BlockSpec((BE, D), lambda i: (i, 0)),
    )(r, wt)


def _tc_q(acc, wt, b, xp):
    def body(acc_ref, w_ref, b_ref, xp_ref, o_ref):
        na = acc_ref[0] + acc_ref[1]
        o_ref[...] = (jnp.dot(na, w_ref[...], preferred_element_type=f32)
                      + b_ref[...] + xp_ref[...])

    return pl.pallas_call(
        body,
        out_shape=jax.ShapeDtypeStruct((N_NODES, D), f32),
        grid=(1,),
        in_specs=[pl.BlockSpec((NC, N_NODES, D), lambda i: (0, 0, 0)),
                  pl.BlockSpec((D, D), lambda i: (0, 0)),
                  pl.BlockSpec((1, D), lambda i: (0, 0)),
                  pl.BlockSpec((N_NODES, D), lambda i: (0, 0))],
        out_specs=pl.BlockSpec((N_NODES, D), lambda i: (0, 0)),
    )(acc, wt, b, xp)


def _tc_final(x, acc, pm, wxt, wmt, b):
    def body(x_ref, acc_ref, pm_ref, wx_ref, wm_ref, b_ref, o_ref):
        m = jnp.dot(acc_ref[0] + acc_ref[1], pm_ref[...],
                    preferred_element_type=f32)
        s = jnp.sum(m, axis=1, keepdims=True)
        m = jnp.where(s == 0.0, x_ref[...], m)
        o_ref[...] = jnp.maximum(
            jnp.dot(x_ref[...], wx_ref[...], preferred_element_type=f32)
            + jnp.dot(m, wm_ref[...], preferred_element_type=f32)
            + b_ref[...], 0.0)

    return pl.pallas_call(
        body,
        out_shape=jax.ShapeDtypeStruct((N_NODES, D), f32),
        grid=(1,),
        in_specs=[pl.BlockSpec((N_NODES, D), lambda i: (0, 0)),
                  pl.BlockSpec((NC, N_NODES, D), lambda i: (0, 0, 0)),
                  pl.BlockSpec((D, D), lambda i: (0, 0)),
                  pl.BlockSpec((D, D), lambda i: (0, 0)),
                  pl.BlockSpec((D, D), lambda i: (0, 0)),
                  pl.BlockSpec((1, D), lambda i: (0, 0))],
        out_specs=pl.BlockSpec((N_NODES, D), lambda i: (0, 0)),
    )(x, acc, pm, wxt, wmt, b)


def kernel(x, edge_index, rev_edge_index, edge_attr, Wi, bi, Wh, bh, Wo, bo):
    src = edge_index[0].astype(jnp.int32)
    dst = edge_index[1].astype(jnp.int32)
    rev = rev_edge_index.astype(jnp.int32)
    perm = jnp.asarray(_PERM_NP)
    P = jnp.asarray(_P_NP)
    WixT = Wi[:, :D].T[:, perm]
    WieT = Wi[:, D:].T
    WhT = Wh.T
    WhTp = WhT[:, perm]
    PWhTp = P @ WhTp
    WoxT = Wo[:, :D].T
    WomT = Wo[:, D:].T
    bi2 = bi.reshape(1, D)
    bh2 = bh.reshape(1, D)
    bo2 = bo.reshape(1, D)
    zeros = jnp.zeros((N_NODES, D), f32)

    bhp = bh[perm].reshape(1, D)

    Xp = _tc_xp(x, WixT)
    Epb = _tc_ep(edge_attr, WieT, bi2)
    Ep = lax.bitcast_convert_type(Epb.reshape(N_EDGES * DW, 2), jnp.int32)
    R, acc = _SC_BUILD(Ep, Xp, src, dst, zeros)

    def r_to_bf16(r):
        return lax.bitcast_convert_type(r, bf16).reshape(N_EDGES, D)

    Q = _tc_q(acc, PWhTp, bhp, Xp)
    G = _tc_g(r_to_bf16(R), WhTp)
    R, acc = _SC_STEP(Ep, Q, G, src, rev, dst, zeros)

    Q = _tc_q(acc, PWhTp, bhp, Xp)
    G = _tc_g(r_to_bf16(R), WhTp)
    acc = _SC_LAST(Ep, Q, G, src, rev, dst, zeros)

    return _tc_final(x, acc, P, WoxT, WomT, bo2)


# speedup vs baseline: 9.2999x; 9.2999x over previous
import functools

import jax
import jax.numpy as jnp
from jax import lax
from jax.experimental import pallas as pl
from jax.experimental.pallas import tpu as pltpu
from jax.experimental.pallas import tpu_sc as plsc

N_NODES = 10000
N_EDGES = 320000
D = 128
D_EDGE = 16
NC = 2
NS = 16
NW = NC * NS
EC = 40
NCHUNK = N_EDGES // EC
CPW = NCHUNK // NW
NROW_CHK = 400
NROW_NCHK = N_NODES // NROW_CHK
BE = 8000

f32 = jnp.float32


def _mesh():
    return plsc.VectorSubcoreMesh(core_axis_name="c", subcore_axis_name="s",
                                  num_cores=NC, num_subcores=NS)


def _build_sc_pass(has_g: bool, write_r: bool):
    acc_t = jax.ShapeDtypeStruct((NC, N_NODES, D), f32)
    out_type = ([jax.ShapeDtypeStruct((N_EDGES, D), f32), acc_t]
                if write_r else acc_t)

    scratch = [pltpu.VMEM((EC,), jnp.int32) for _ in range(2)]
    if has_g:
        scratch += [pltpu.VMEM((EC,), jnp.int32) for _ in range(2)]
    scratch += [pltpu.VMEM((EC,), jnp.int32) for _ in range(2)]
    scratch += [pltpu.VMEM((EC, D), f32) for _ in range(2)]
    scratch += [pltpu.VMEM((EC, D), f32) for _ in range(2)]
    if has_g:
        scratch += [pltpu.VMEM((EC, D), f32) for _ in range(2)]
    scratch.append(pltpu.VMEM_SHARED((N_NODES, D), f32))
    scratch += [pltpu.SemaphoreType.DMA for _ in range(8)]

    @functools.partial(pl.kernel, out_type=out_type, mesh=_mesh(),
                       scratch_types=scratch)
    def k(*args):
        it = iter(args)
        ep = next(it)
        qt = next(it)
        gt = next(it) if has_g else None
        src = next(it)
        rev = next(it) if has_g else None
        dst = next(it)
        zeros = next(it)
        r_out = next(it) if write_r else None
        acc = next(it)
        sibuf = [next(it), next(it)]
        ribuf = [next(it), next(it)] if has_g else None
        dbuf = [next(it), next(it)]
        eprow = [next(it), next(it)]
        qrow = [next(it), next(it)]
        grow = [next(it), next(it)] if has_g else None
        shacc = next(it)
        sem_idx = [next(it), next(it)]
        sem_rows = [next(it), next(it)]
        sem_sc = [next(it), next(it)]
        sem_st = [next(it), next(it)]

        cid = lax.axis_index("c")
        sid = lax.axis_index("s")
        wid = sid * NC + cid

        nlo = sid * NROW_NCHK // NS
        nhi = (sid + 1) * NROW_NCHK // NS

        def zero_chunk(kk, carry):
            rr = kk * NROW_CHK
            pltpu.sync_copy(zeros.at[pl.ds(rr, NROW_CHK)],
                            shacc.at[pl.ds(rr, NROW_CHK)])
            return carry

        lax.fori_loop(nlo, nhi, zero_chunk, 0)
        plsc.subcore_barrier()

        ebase = wid * CPW

        def issue_gidx(t, p):
            b = (ebase + t) * EC
            pltpu.async_copy(src.at[pl.ds(b, EC)], sibuf[p], sem_idx[p])
            if has_g:
                pltpu.async_copy(rev.at[pl.ds(b, EC)], ribuf[p], sem_idx[p])

        def issue_didx(t, p):
            b = (ebase + t) * EC
            pltpu.async_copy(dst.at[pl.ds(b, EC)], dbuf[p], sem_idx[p])

        def wait_idx(p):
            pltpu.make_async_copy(src.at[pl.ds(0, EC)], sibuf[p],
                                  sem_idx[p]).wait()
            if has_g:
                pltpu.make_async_copy(src.at[pl.ds(0, EC)], ribuf[p],
                                      sem_idx[p]).wait()
            pltpu.make_async_copy(src.at[pl.ds(0, EC)], dbuf[p],
                                  sem_idx[p]).wait()

        def issue_rows(t, p):
            b = (ebase + t) * EC
            de = pltpu.async_copy(ep.at[pl.ds(b, EC)], eprow[p], sem_rows[p])
            dq = pltpu.async_copy(qt.at[sibuf[p]], qrow[p], sem_rows[p])
            dg = (pltpu.async_copy(gt.at[ribuf[p]], grow[p], sem_rows[p])
                  if has_g else None)
            return de, dq, dg

        def compute(p):
            def row(e, cc):
                for j in range(D // 16):
                    sl = pl.ds(j * 16, 16)
                    v = eprow[p][e, sl] + qrow[p][e, sl]
                    if has_g:
                        v = v - grow[p][e, sl]
                    qrow[p][e, sl] = jnp.maximum(v, 0.0)
                return cc

            lax.fori_loop(0, EC, row, 0)

        def issue_outs(t, p):
            dsc = pltpu.async_copy(qrow[p], shacc.at[dbuf[p]], sem_sc[p],
                                   add=True)
            if write_r:
                b = (ebase + t) * EC
                pltpu.async_copy(qrow[p], r_out.at[pl.ds(b, EC)], sem_st[p])
            return dsc

        def wait_store(p):
            if write_r:
                pltpu.make_async_copy(ep.at[pl.ds(0, EC)], qrow[p],
                                      sem_st[p]).wait()

        def finish(t, p):
            compute(p)
            dsc = issue_outs(t, p)
            return dsc

        b0 = ebase * EC
        pltpu.sync_copy(src.at[pl.ds(b0, EC)], sibuf[0])
        if has_g:
            pltpu.sync_copy(rev.at[pl.ds(b0, EC)], ribuf[0])
        pltpu.sync_copy(dst.at[pl.ds(b0, EC)], dbuf[0])
        de, dq, dg = issue_rows(0, 0)
        issue_gidx(1, 1)
        de.wait()
        dq.wait()
        if has_g:
            dg.wait()
        issue_didx(1, 1)
        wait_idx(1)
        de1, dq1, dg1 = issue_rows(1, 1)
        issue_gidx(2, 0)
        dsc0 = finish(0, 0)
        dsc0.wait()
        de1.wait()
        dq1.wait()
        if has_g:
            dg1.wait()
        issue_didx(2, 0)

        def body(m, carry):
            t0 = 2 * m
            t1 = t0 + 1

            wait_idx(0)
            wait_store(0)
            de, dq, dg = issue_rows(t0, 0)
            issue_gidx(t0 + 1, 1)
            dsc = finish(t0 - 1, 1)
            de.wait()
            dq.wait()
            if has_g:
                dg.wait()
            dsc.wait()
            issue_didx(t0 + 1, 1)

            wait_idx(1)
            wait_store(1)
            de1, dq1, dg1 = issue_rows(t1, 1)

            @pl.when(m < CPW // 2 - 1)
            def _i1():
                issue_gidx(t1 + 1, 0)

            dsc0 = finish(t0, 0)
            de1.wait()
            dq1.wait()
            if has_g:
                dg1.wait()
            dsc0.wait()

            @pl.when(m < CPW // 2 - 1)
            def _i2():
                issue_didx(t1 + 1, 0)

            return carry

        lax.fori_loop(1, CPW // 2, body, 0)

        dsc = finish(CPW - 1, 1)
        dsc.wait()
        wait_store(0)
        wait_store(1)

        plsc.subcore_barrier()

        def dump_chunk(kk, carry):
            rr = kk * NROW_CHK
            pltpu.sync_copy(shacc.at[pl.ds(rr, NROW_CHK)],
                            acc.at[cid, pl.ds(rr, NROW_CHK)])
            return carry

        lax.fori_loop(nlo, nhi, dump_chunk, 0)

    return k


_SC_BUILD = _build_sc_pass(has_g=False, write_r=True)
_SC_STEP = _build_sc_pass(has_g=True, write_r=True)
_SC_LAST = _build_sc_pass(has_g=True, write_r=False)


def _tc_xp(x, wt):
    def body(x_ref, w_ref, o_ref):
        o_ref[...] = jnp.dot(x_ref[...], w_ref[...], preferred_element_type=f32)

    return pl.pallas_call(
        body,
        out_shape=jax.ShapeDtypeStruct((N_NODES, D), f32),
        grid=(1,),
        in_specs=[pl.BlockSpec((N_NODES, D), lambda i: (0, 0)),
                  pl.BlockSpec((D, D), lambda i: (0, 0))],
        out_specs=pl.BlockSpec((N_NODES, D), lambda i: (0, 0)),
    )(x, wt)


def _tc_ep(ea, wt, b):
    def body(e_ref, w_ref, b_ref, o_ref):
        o_ref[...] = (jnp.dot(e_ref[...], w_ref[...], preferred_element_type=f32)
                      + b_ref[...])

    return pl.pallas_call(
        body,
        out_shape=jax.ShapeDtypeStruct((N_EDGES, D), f32),
        grid=(N_EDGES // BE,),
        in_specs=[pl.BlockSpec((BE, D_EDGE), lambda i: (i, 0)),
                  pl.BlockSpec((D_EDGE, D), lambda i: (0, 0)),
                  pl.BlockSpec((1, D), lambda i: (0, 0))],
        out_specs=pl.BlockSpec((BE, D), lambda i: (i, 0)),
    )(ea, wt, b)


def _tc_g(r, wt):
    def body(r_ref, w_ref, o_ref):
        o_ref[...] = jnp.dot(r_ref[...], w_ref[...], preferred_element_type=f32)

    return pl.pallas_call(
        body,
        out_shape=jax.ShapeDtypeStruct((N_EDGES, D), f32),
        grid=(N_EDGES // BE,),
        in_specs=[pl.BlockSpec((BE, D), lambda i: (i, 0)),
                  pl.BlockSpec((D, D), lambda i: (0, 0))],
        out_specs=pl.BlockSpec((BE, D), lambda i: (i, 0)),
    )(r, wt)


def _tc_q(acc, wt, b, xp):
    def body(acc_ref, w_ref, b_ref, xp_ref, o_ref):
        na = acc_ref[0] + acc_ref[1]
        o_ref[...] = (jnp.dot(na, w_ref[...], preferred_element_type=f32)
                      + b_ref[...] + xp_ref[...])

    return pl.pallas_call(
        body,
        out_shape=jax.ShapeDtypeStruct((N_NODES, D), f32),
        grid=(1,),
        in_specs=[pl.BlockSpec((NC, N_NODES, D), lambda i: (0, 0, 0)),
                  pl.BlockSpec((D, D), lambda i: (0, 0)),
                  pl.BlockSpec((1, D), lambda i: (0, 0)),
                  pl.BlockSpec((N_NODES, D), lambda i: (0, 0))],
        out_specs=pl.BlockSpec((N_NODES, D), lambda i: (0, 0)),
    )(acc, wt, b, xp)


def _tc_final(x, acc, wxt, wmt, b):
    def body(x_ref, acc_ref, wx_ref, wm_ref, b_ref, o_ref):
        m = acc_ref[0] + acc_ref[1]
        s = jnp.sum(m, axis=1, keepdims=True)
        m = jnp.where(s == 0.0, x_ref[...], m)
        o_ref[...] = jnp.maximum(
            jnp.dot(x_ref[...], wx_ref[...], preferred_element_type=f32)
            + jnp.dot(m, wm_ref[...], preferred_element_type=f32)
            + b_ref[...], 0.0)

    return pl.pallas_call(
        body,
        out_shape=jax.ShapeDtypeStruct((N_NODES, D), f32),
        grid=(1,),
        in_specs=[pl.BlockSpec((N_NODES, D), lambda i: (0, 0)),
                  pl.BlockSpec((NC, N_NODES, D), lambda i: (0, 0, 0)),
                  pl.BlockSpec((D, D), lambda i: (0, 0)),
                  pl.BlockSpec((D, D), lambda i: (0, 0)),
                  pl.BlockSpec((1, D), lambda i: (0, 0))],
        out_specs=pl.BlockSpec((N_NODES, D), lambda i: (0, 0)),
    )(x, acc, wxt, wmt, b)


def kernel(x, edge_index, rev_edge_index, edge_attr, Wi, bi, Wh, bh, Wo, bo):
    src = edge_index[0].astype(jnp.int32)
    dst = edge_index[1].astype(jnp.int32)
    rev = rev_edge_index.astype(jnp.int32)
    WixT = Wi[:, :D].T
    WieT = Wi[:, D:].T
    WhT = Wh.T
    WoxT = Wo[:, :D].T
    WomT = Wo[:, D:].T
    bi2 = bi.reshape(1, D)
    bh2 = bh.reshape(1, D)
    bo2 = bo.reshape(1, D)
    zeros = jnp.zeros((N_NODES, D), f32)

    Xp = _tc_xp(x, WixT)
    Ep = _tc_ep(edge_attr, WieT, bi2)
    R, acc = _SC_BUILD(Ep, Xp, src, dst, zeros)

    Q = _tc_q(acc, WhT, bh2, Xp)
    G = _tc_g(R, WhT)
    R, acc = _SC_STEP(Ep, Q, G, src, rev, dst, zeros)

    Q = _tc_q(acc, WhT, bh2, Xp)
    G = _tc_g(R, WhT)
    acc = _SC_LAST(Ep, Q, G, src, rev, dst, zeros)

    return _tc_final(x, acc, WoxT, WomT, bo2)
